# baseline (speedup 1.0000x reference)
import jax
import jax.numpy as jnp
from jax.experimental import pallas as pl
from jax.experimental.pallas import tpu as pltpu

_NC = 4


def _gru_fused_kernel(ids_ref, emb_hbm_ref, w_ih_ref, w_hh_ref, bias_ref,
                      b_hn_ref, out_ref, hid_ref,
                      emb_vm_ref, gi_ref, h_ref, sems):
    S = emb_hbm_ref.shape[1]
    B, Tc, H = out_ref.shape
    H2 = 2 * H
    R = Tc * B
    c = pl.program_id(0)
    nc = pl.num_programs(0)

    def issue_chunk(chunk):
        base = chunk * R

        def body(k, _):
            row = base + k
            idx = ids_ref[row]
            pltpu.make_async_copy(
                emb_hbm_ref.at[idx],
                emb_vm_ref.at[:, row, :],
                sems.at[chunk]).start()
            return _

        jax.lax.fori_loop(0, R, body, None)

    @pl.when(c == 0)
    def _issue_first():
        issue_chunk(0)

    @pl.when(c < nc - 1)
    def _issue_next():
        issue_chunk(c + 1)

    @pl.when(c == 0)
    def _init():
        h_ref[...] = jnp.zeros_like(h_ref)

    slab = emb_vm_ref.at[:, pl.ds(c * R, R), :]
    pltpu.make_async_copy(slab, slab, sems.at[c]).wait()

    emb_c = jnp.concatenate(
        [emb_vm_ref[s, pl.ds(c * R, R), :] for s in range(S)],
        axis=1).astype(jnp.bfloat16)
    w_ih_full = w_ih_ref[...].reshape(S * 128, 3 * H)
    gi = jnp.dot(emb_c, w_ih_full, preferred_element_type=jnp.float32)
    gi_ref[...] = gi + bias_ref[...]

    b_hn = jnp.broadcast_to(b_hn_ref[...], (B, H))
    h = h_ref[...]

    for t in range(Tc):
        gi_t = gi_ref[t * B:(t + 1) * B]
        h_b = h.astype(jnp.bfloat16)

        gh_rz = jnp.dot(h_b, w_hh_ref[:, 0:H2],
                        preferred_element_type=jnp.float32)
        r = jax.nn.sigmoid(gi_t[:, 0:H] + gh_rz[:, 0:H])
        z = jax.nn.sigmoid(gi_t[:, H:H2] + gh_rz[:, H:H2])

        gh_n = jnp.dot(h_b, w_hh_ref[:, H2:],
                       preferred_element_type=jnp.float32)
        n = jnp.tanh(gi_t[:, H2:] + r * (gh_n + b_hn))

        h = n + z * (h - n)
        out_ref[:, t, :] = h

    h_ref[...] = h
    hid_ref[...] = h


def kernel(x_ids, emb_table, w_ih, w_hh, b_ih, b_hh):
    B, T = x_ids.shape
    V, H = emb_table.shape
    S = H // 128
    nc = _NC if T % _NC == 0 else 1
    tc = T // nc

    ids_tm = x_ids.T.reshape(T * B)
    emb_rows = emb_table.reshape(V, S, 128)

    w_ih_t = w_ih.T.astype(jnp.bfloat16).reshape(S, 128, 3 * H)
    w_hh_t = w_hh.T.astype(jnp.bfloat16)
    b_rz = jnp.concatenate([b_hh[:2 * H], jnp.zeros((H,), b_hh.dtype)])
    bias = (b_ih + b_rz).reshape(1, 3 * H)
    b_hn = b_hh[2 * H:].reshape(1, H)

    output, hidden = pl.pallas_call(
        _gru_fused_kernel,
        out_shape=(
            jax.ShapeDtypeStruct((B, T, H), jnp.float32),
            jax.ShapeDtypeStruct((B, H), jnp.float32),
        ),
        grid_spec=pltpu.PrefetchScalarGridSpec(
            num_scalar_prefetch=1,
            grid=(nc,),
            in_specs=[
                pl.BlockSpec(memory_space=pl.ANY),
                pl.BlockSpec((S, 128, 3 * H), lambda c, ids: (0, 0, 0)),
                pl.BlockSpec((H, 3 * H), lambda c, ids: (0, 0)),
                pl.BlockSpec((1, 3 * H), lambda c, ids: (0, 0)),
                pl.BlockSpec((1, H), lambda c, ids: (0, 0)),
            ],
            out_specs=(
                pl.BlockSpec((B, tc, H), lambda c, ids: (0, c, 0)),
                pl.BlockSpec((B, H), lambda c, ids: (0, 0)),
            ),
            scratch_shapes=[
                pltpu.VMEM((S, T * B, 128), jnp.float32),
                pltpu.VMEM((tc * B, 3 * H), jnp.float32),
                pltpu.VMEM((B, H), jnp.float32),
                pltpu.SemaphoreType.DMA((_NC,)),
            ],
        ),
        compiler_params=pltpu.CompilerParams(
            dimension_semantics=("arbitrary",)),
    )(ids_tm, emb_rows, w_ih_t, w_hh_t, bias, b_hn)

    return output, hidden.reshape(1, B, H)


# speedup vs baseline: 1.0027x; 1.0027x over previous
import jax
import jax.numpy as jnp
from jax.experimental import pallas as pl
from jax.experimental.pallas import tpu as pltpu

_NC = 4


def _gru_fused_kernel(ids_ref, emb_hbm_ref, w_ih_ref, w_hh_ref, bias_ref,
                      b_hn_ref, out_ref, hid_ref,
                      emb_vm_ref, gi_ref, h_ref, sems):
    S = emb_hbm_ref.shape[1]
    B, Tc, H = out_ref.shape
    H2 = 2 * H
    R = Tc * B
    c = pl.program_id(0)
    nc = pl.num_programs(0)

    def issue_chunk(chunk):
        base = chunk * R

        def body(k, _):
            row = base + k
            idx = ids_ref[row]
            pltpu.make_async_copy(
                emb_hbm_ref.at[idx],
                emb_vm_ref.at[:, row, :],
                sems.at[chunk]).start()
            return _

        jax.lax.fori_loop(0, R, body, None)

    @pl.when(c == 0)
    def _issue_first():
        issue_chunk(0)

    @pl.when(c < nc - 1)
    def _issue_next():
        issue_chunk(c + 1)

    @pl.when(c == 0)
    def _init():
        h_ref[...] = jnp.zeros_like(h_ref)

    slab = emb_vm_ref.at[:, pl.ds(c * R, R), :]
    pltpu.make_async_copy(slab, slab, sems.at[c]).wait()

    emb_c = jnp.concatenate(
        [emb_vm_ref[s, pl.ds(c * R, R), :] for s in range(S)],
        axis=1).astype(jnp.bfloat16)
    w_ih_full = w_ih_ref[...].reshape(S * 128, 3 * H)
    gi = jnp.dot(emb_c, w_ih_full, preferred_element_type=jnp.float32)
    gi_ref[...] = gi + bias_ref[...]

    b_hn = jnp.broadcast_to(b_hn_ref[...], (B, H))
    h = h_ref[...]

    for t in range(Tc):
        gi_t = gi_ref[t * B:(t + 1) * B]
        h_b = h.astype(jnp.bfloat16)

        gh_rz = jnp.dot(h_b, w_hh_ref[:, 0:H2],
                        preferred_element_type=jnp.float32)
        r = jax.nn.sigmoid(gi_t[:, 0:H] + gh_rz[:, 0:H])
        z = jax.nn.sigmoid(gi_t[:, H:H2] + gh_rz[:, H:H2])

        gh_n = jnp.dot(h_b, w_hh_ref[:, H2:],
                       preferred_element_type=jnp.float32)
        n = jnp.tanh(gi_t[:, H2:] + r * (gh_n + b_hn))

        h = n + z * (h - n)
        out_ref[:, t, :] = h

    h_ref[...] = h
    hid_ref[...] = h


def kernel(x_ids, emb_table, w_ih, w_hh, b_ih, b_hh):
    B, T = x_ids.shape
    V, H = emb_table.shape
    S = H // 128
    nc = _NC if T % _NC == 0 else 1
    tc = T // nc

    ids_tm = x_ids.T.reshape(T * B)
    emb_rows = emb_table.reshape(V, S, 128)

    w_ih_t = w_ih.T.astype(jnp.bfloat16).reshape(S, 128, 3 * H)
    w_hh_t = w_hh.T.astype(jnp.bfloat16)
    b_rz = jnp.concatenate([b_hh[:2 * H], jnp.zeros((H,), b_hh.dtype)])
    bias = (b_ih + b_rz).reshape(1, 3 * H)
    b_hn = b_hh[2 * H:].reshape(1, H)

    output, hidden = pl.pallas_call(
        _gru_fused_kernel,
        out_shape=(
            jax.ShapeDtypeStruct((B, T, H), jnp.float32),
            jax.ShapeDtypeStruct((B, H), jnp.float32),
        ),
        grid_spec=pltpu.PrefetchScalarGridSpec(
            num_scalar_prefetch=1,
            grid=(nc,),
            in_specs=[
                pl.BlockSpec(memory_space=pl.ANY),
                pl.BlockSpec((S, 128, 3 * H), lambda c, ids: (0, 0, 0)),
                pl.BlockSpec((H, 3 * H), lambda c, ids: (0, 0)),
                pl.BlockSpec((1, 3 * H), lambda c, ids: (0, 0)),
                pl.BlockSpec((1, H), lambda c, ids: (0, 0)),
            ],
            out_specs=(
                pl.BlockSpec((B, tc, H), lambda c, ids: (0, c, 0)),
                pl.BlockSpec((B, H), lambda c, ids: (0, 0)),
            ),
            scratch_shapes=[
                pltpu.VMEM((S, T * B, 128), jnp.float32),
                pltpu.VMEM((tc * B, 3 * H), jnp.float32),
                pltpu.VMEM((B, H), jnp.float32),
                pltpu.SemaphoreType.DMA((_NC,)),
            ],
        ),
        compiler_params=pltpu.CompilerParams(
            dimension_semantics=("arbitrary",),
            disable_bounds_checks=True),
    )(ids_tm, emb_rows, w_ih_t, w_hh_t, bias, b_hn)

    return output, hidden.reshape(1, B, H)


# speedup vs baseline: 1.7657x; 1.7610x over previous
import jax
import jax.numpy as jnp
from jax.experimental import pallas as pl
from jax.experimental.pallas import tpu as pltpu

_NC = 2


def _gru_fused_kernel(emb_ref, w_ih_ref, w_hh_ref, bias_ref, b_hn_ref,
                      out_ref, hid_ref, gi_ref, h_ref):
    Tc, B, H = emb_ref.shape
    H2 = 2 * H
    c = pl.program_id(0)

    gi = jax.lax.dot_general(
        emb_ref[...].astype(jnp.bfloat16), w_ih_ref[...],
        dimension_numbers=(((2,), (0,)), ((), ())),
        preferred_element_type=jnp.float32)
    gi_ref[...] = gi + bias_ref[...]

    @pl.when(c == 0)
    def _init():
        h_ref[...] = jnp.zeros_like(h_ref)

    b_hn = jnp.broadcast_to(b_hn_ref[...], (B, H))
    h = h_ref[...]

    for t in range(Tc):
        gi_t = gi_ref[t]
        h_b = h.astype(jnp.bfloat16)

        gh_rz = jnp.dot(h_b, w_hh_ref[:, 0:H2],
                        preferred_element_type=jnp.float32)
        r = jax.nn.sigmoid(gi_t[:, 0:H] + gh_rz[:, 0:H])
        z = jax.nn.sigmoid(gi_t[:, H:H2] + gh_rz[:, H:H2])

        gh_n = jnp.dot(h_b, w_hh_ref[:, H2:],
                       preferred_element_type=jnp.float32)
        n = jnp.tanh(gi_t[:, H2:] + r * (gh_n + b_hn))

        h = n + z * (h - n)
        out_ref[:, t, :] = h

    h_ref[...] = h
    hid_ref[...] = h


def kernel(x_ids, emb_table, w_ih, w_hh, b_ih, b_hh):
    B, T = x_ids.shape
    H = emb_table.shape[1]
    nc = _NC if T % _NC == 0 else 1
    tc = T // nc

    embedded_tm = emb_table[x_ids.T]

    w_ih_t = w_ih.T.astype(jnp.bfloat16)
    w_hh_t = w_hh.T.astype(jnp.bfloat16)
    b_rz = jnp.concatenate([b_hh[:2 * H], jnp.zeros((H,), b_hh.dtype)])
    bias = (b_ih + b_rz).reshape(1, 3 * H)
    b_hn = b_hh[2 * H:].reshape(1, H)

    output, hidden = pl.pallas_call(
        _gru_fused_kernel,
        out_shape=(
            jax.ShapeDtypeStruct((B, T, H), jnp.float32),
            jax.ShapeDtypeStruct((B, H), jnp.float32),
        ),
        grid=(nc,),
        in_specs=[
            pl.BlockSpec((tc, B, H), lambda c: (c, 0, 0)),
            pl.BlockSpec((H, 3 * H), lambda c: (0, 0)),
            pl.BlockSpec((H, 3 * H), lambda c: (0, 0)),
            pl.BlockSpec((1, 3 * H), lambda c: (0, 0)),
            pl.BlockSpec((1, H), lambda c: (0, 0)),
        ],
        out_specs=(
            pl.BlockSpec((B, tc, H), lambda c: (0, c, 0)),
            pl.BlockSpec((B, H), lambda c: (0, 0)),
        ),
        scratch_shapes=[
            pltpu.VMEM((tc, B, 3 * H), jnp.float32),
            pltpu.VMEM((B, H), jnp.float32),
        ],
        compiler_params=pltpu.CompilerParams(
            dimension_semantics=("arbitrary",)),
    )(embedded_tm, w_ih_t, w_hh_t, bias, b_hn)

    return output, hidden.reshape(1, B, H)


# speedup vs baseline: 1.7879x; 1.0126x over previous
import jax
import jax.numpy as jnp
from jax.experimental import pallas as pl
from jax.experimental.pallas import tpu as pltpu

_NC = 4


def _gru_fused_kernel(emb_ref, w_ih_ref, w_hh_ref, bias_ref, b_hn_ref,
                      out_ref, hid_ref, gi_ref, h_ref):
    Tc, B, H = emb_ref.shape
    H2 = 2 * H
    c = pl.program_id(0)

    gi = jax.lax.dot_general(
        emb_ref[...].astype(jnp.bfloat16), w_ih_ref[...],
        dimension_numbers=(((2,), (0,)), ((), ())),
        preferred_element_type=jnp.float32)
    gi_ref[...] = gi + bias_ref[...]

    @pl.when(c == 0)
    def _init():
        h_ref[...] = jnp.zeros_like(h_ref)

    b_hn = jnp.broadcast_to(b_hn_ref[...], (B, H))
    h = h_ref[...]

    for t in range(Tc):
        gi_t = gi_ref[t]
        h_b = h.astype(jnp.bfloat16)

        gh = jnp.dot(h_b, w_hh_ref[...],
                     preferred_element_type=jnp.float32)
        r = jax.nn.sigmoid(gi_t[:, 0:H] + gh[:, 0:H])
        z = jax.nn.sigmoid(gi_t[:, H:H2] + gh[:, H:H2])
        n = jnp.tanh(gi_t[:, H2:] + r * (gh[:, H2:] + b_hn))

        h = n + z * (h - n)
        out_ref[:, t, :] = h

    h_ref[...] = h
    hid_ref[...] = h


def kernel(x_ids, emb_table, w_ih, w_hh, b_ih, b_hh):
    B, T = x_ids.shape
    H = emb_table.shape[1]
    nc = _NC if T % _NC == 0 else 1
    tc = T // nc

    embedded_tm = emb_table[x_ids.T]

    w_ih_t = w_ih.T.astype(jnp.bfloat16)
    w_hh_t = w_hh.T.astype(jnp.bfloat16)
    b_rz = jnp.concatenate([b_hh[:2 * H], jnp.zeros((H,), b_hh.dtype)])
    bias = (b_ih + b_rz).reshape(1, 3 * H)
    b_hn = b_hh[2 * H:].reshape(1, H)

    output, hidden = pl.pallas_call(
        _gru_fused_kernel,
        out_shape=(
            jax.ShapeDtypeStruct((B, T, H), jnp.float32),
            jax.ShapeDtypeStruct((B, H), jnp.float32),
        ),
        grid=(nc,),
        in_specs=[
            pl.BlockSpec((tc, B, H), lambda c: (c, 0, 0)),
            pl.BlockSpec((H, 3 * H), lambda c: (0, 0)),
            pl.BlockSpec((H, 3 * H), lambda c: (0, 0)),
            pl.BlockSpec((1, 3 * H), lambda c: (0, 0)),
            pl.BlockSpec((1, H), lambda c: (0, 0)),
        ],
        out_specs=(
            pl.BlockSpec((B, tc, H), lambda c: (0, c, 0)),
            pl.BlockSpec((B, H), lambda c: (0, 0)),
        ),
        scratch_shapes=[
            pltpu.VMEM((tc, B, 3 * H), jnp.float32),
            pltpu.VMEM((B, H), jnp.float32),
        ],
        compiler_params=pltpu.CompilerParams(
            dimension_semantics=("arbitrary",)),
    )(embedded_tm, w_ih_t, w_hh_t, bias, b_hn)

    return output, hidden.reshape(1, B, H)
